# baseline (speedup 1.0000x reference)
import functools

import numpy as np
import jax
import jax.numpy as jnp
from jax import lax
from jax.experimental import pallas as pl
from jax.experimental.pallas import tpu as pltpu
from jax.experimental.pallas import tpu_sc as plsc

_NUM_VN = 6
_INV2T2 = 50.0
_BIG = 3.0e38

_NPAD = 30720
_WPTS = 960
_CH = 480
_NV = _CH // 16


def _transforms_kernel(x4_ref, w0, w1, w2, w3, w4, w5, out_ref):
    x = x4_ref[...]
    feats = [x]
    for f in (1.0, 2.0):
        feats.append(jnp.sin(x * f))
        feats.append(jnp.cos(x * f))
    h = jnp.concatenate(feats, axis=-1)
    for w in (w0, w1, w2, w3, w4):
        h = jnp.maximum(jnp.dot(h, w[...], preferred_element_type=jnp.float32), 0.0)
    o = jnp.dot(h, w5[...], preferred_element_type=jnp.float32)
    ax, ay, az = o[:, 0:1], o[:, 1:2], o[:, 2:3]
    x2, y2, z2 = ax * ax, ay * ay, az * az
    theta2 = x2 + y2 + z2
    theta = jnp.sqrt(theta2 + 1e-12)
    A = jnp.sin(theta) / theta
    B = (1.0 - jnp.cos(theta)) / (theta2 + 1e-12)
    xy, xz, yz = ax * ay, ax * az, ay * az
    r00 = 1.0 - B * (y2 + z2)
    r01 = -A * az + B * xy
    r02 = A * ay + B * xz
    r10 = A * az + B * xy
    r11 = 1.0 - B * (x2 + z2)
    r12 = -A * ax + B * yz
    r20 = -A * ay + B * xz
    r21 = A * ax + B * yz
    r22 = 1.0 - B * (x2 + y2)
    out_ref[...] = jnp.concatenate(
        [r00, r01, r02, r10, r11, r12, r20, r21, r22,
         o[:, 3:4], o[:, 4:5], o[:, 5:6]], axis=-1)


def _knn_kernel(ppt_ref, c2_ref, idx_ref, w_ref):
    e = jnp.dot(c2_ref[...], ppt_ref[...], preferred_element_type=jnp.float32)
    ncp, P = e.shape
    riota = lax.broadcasted_iota(jnp.int32, (ncp, P), 0).astype(jnp.float32)
    ew = e
    m1 = None
    idx_rows = []
    w_rows = []
    for k in range(_NUM_VN):
        mk = jnp.min(ew, axis=0, keepdims=True)
        if k == 0:
            m1 = mk
        t = jnp.where(ew <= mk, riota, 1.0e9)
        ik = jnp.min(t, axis=0, keepdims=True)
        idx_rows.append(ik)
        w_rows.append(jnp.exp((m1 - mk) * _INV2T2))
        if k < _NUM_VN - 1:
            ew = jnp.where(t == ik, _BIG, ew)
    z = w_rows[0]
    for k in range(1, _NUM_VN):
        z = z + w_rows[k]
    zero = jnp.zeros_like(m1)
    w8 = jnp.concatenate(w_rows + [zero, zero], axis=0) / z
    idx8 = jnp.concatenate(idx_rows + [zero, zero], axis=0)
    idx_ref[...] = idx8.astype(jnp.int32)
    w_ref[...] = w8


def _gsum(tab_v, ib, wk, comp):
    r = None
    for k in range(_NUM_VN):
        g = plsc.load_gather(tab_v, [ib[k] + comp])
        r = g * wk[k] if r is None else r + g * wk[k]
    return r


def _sc_blend_body(tab_hbm, idx_hbm, w_hbm, p_hbm, out_hbm,
                   tab_v, idx_v, w_v, p_v, out_v):
    wid = lax.axis_index("s") * 2 + lax.axis_index("c")
    pltpu.sync_copy(tab_hbm, tab_v)
    for cc in range(_WPTS // _CH):
        base = wid * _WPTS + cc * _CH
        for k in range(_NUM_VN):
            pltpu.sync_copy(idx_hbm.at[pl.ds(k * _NPAD + base, _CH)],
                            idx_v.at[pl.ds(k * _CH, _CH)])
            pltpu.sync_copy(w_hbm.at[pl.ds(k * _NPAD + base, _CH)],
                            w_v.at[pl.ds(k * _CH, _CH)])
        for d in range(3):
            pltpu.sync_copy(p_hbm.at[pl.ds(d * _NPAD + base, _CH)],
                            p_v.at[pl.ds(d * _CH, _CH)])

        def body(i, carry):
            off = i * 16
            ib = [idx_v[pl.ds(k * _CH + off, 16)] * 96 for k in range(_NUM_VN)]
            wk = [w_v[pl.ds(k * _CH + off, 16)] for k in range(_NUM_VN)]
            pv = [p_v[pl.ds(d * _CH + off, 16)] for d in range(3)]
            for b in range(8):
                for c3 in range(3):
                    r = _gsum(tab_v, ib, wk, b * 12 + 9 + c3)
                    for dd in range(3):
                        r = r + _gsum(tab_v, ib, wk, b * 12 + 3 * c3 + dd) * pv[dd]
                    out_v[pl.ds((b * 3 + c3) * _CH + off, 16)] = r
            return carry

        lax.fori_loop(0, _NV, body, 0)
        for c24 in range(24):
            pltpu.sync_copy(out_v.at[pl.ds(c24 * _CH, _CH)],
                            out_hbm.at[pl.ds(c24 * _NPAD + base, _CH)])


def _make_sc_blend():
    return functools.partial(
        pl.kernel,
        mesh=plsc.VectorSubcoreMesh(core_axis_name="c", subcore_axis_name="s"),
        out_type=jax.ShapeDtypeStruct((24 * _NPAD,), jnp.float32),
        compiler_params=pltpu.CompilerParams(needs_layout_passes=False),
        scratch_types=[
            pltpu.VMEM((96000,), jnp.float32),
            pltpu.VMEM((_NUM_VN * _CH,), jnp.int32),
            pltpu.VMEM((_NUM_VN * _CH,), jnp.float32),
            pltpu.VMEM((3 * _CH,), jnp.float32),
            pltpu.VMEM((24 * _CH,), jnp.float32),
        ],
    )(_sc_blend_body)


def kernel(p, t, control_points, W0, W1, W2, W3, W4, W5):
    n = p.shape[0]
    b = t.shape[0]
    ncp = control_points.shape[0]

    tcol = jnp.repeat(t, ncp)[:, None]
    cps = jnp.tile(control_points, (b, 1))
    x4 = jnp.concatenate([tcol, cps], axis=1)
    R = 1000
    tab12 = pl.pallas_call(
        _transforms_kernel,
        grid=(b * ncp // R,),
        in_specs=[
            pl.BlockSpec((R, 4), lambda i: (i, 0)),
            pl.BlockSpec(W0.shape, lambda i: (0, 0)),
            pl.BlockSpec(W1.shape, lambda i: (0, 0)),
            pl.BlockSpec(W2.shape, lambda i: (0, 0)),
            pl.BlockSpec(W3.shape, lambda i: (0, 0)),
            pl.BlockSpec(W4.shape, lambda i: (0, 0)),
            pl.BlockSpec(W5.shape, lambda i: (0, 0)),
        ],
        out_specs=pl.BlockSpec((R, 12), lambda i: (i, 0)),
        out_shape=jax.ShapeDtypeStruct((b * ncp, 12), jnp.float32),
    )(x4, W0, W1, W2, W3, W4, W5)
    table = tab12.reshape(b, ncp, 12).transpose(1, 0, 2).reshape(ncp, b * 12)

    pad = _NPAD - n
    pt_pad = jnp.pad(p.T, ((0, 0), (0, pad)))
    ppt = jnp.concatenate(
        [pt_pad, jnp.ones((1, _NPAD), jnp.float32),
         jnp.zeros((4, _NPAD), jnp.float32)],
        axis=0)
    csq = jnp.sum(control_points * control_points, axis=1, keepdims=True)
    C2 = jnp.concatenate(
        [-2.0 * control_points, csq, jnp.zeros((ncp, 4), jnp.float32)],
        axis=1)

    P = 1536
    idx8, w8 = pl.pallas_call(
        _knn_kernel,
        grid=(_NPAD // P,),
        in_specs=[
            pl.BlockSpec((8, P), lambda i: (0, i)),
            pl.BlockSpec((ncp, 8), lambda i: (0, 0)),
        ],
        out_specs=[
            pl.BlockSpec((8, P), lambda i: (0, i)),
            pl.BlockSpec((8, P), lambda i: (0, i)),
        ],
        out_shape=[
            jax.ShapeDtypeStruct((8, _NPAD), jnp.int32),
            jax.ShapeDtypeStruct((8, _NPAD), jnp.float32),
        ],
    )(ppt, C2)

    idx_flat = idx8.reshape(-1)
    w_flat = w8.reshape(-1)
    p_flat = pt_pad.reshape(-1)
    tab_flat = table.reshape(-1)
    out24 = _make_sc_blend()(tab_flat, idx_flat, w_flat, p_flat)
    out = out24.reshape(24, _NPAD)[:, :n]
    return out.reshape(b, 3, n).transpose(0, 2, 1)


# speedup vs baseline: 1.7077x; 1.7077x over previous
import functools

import numpy as np
import jax
import jax.numpy as jnp
from jax import lax
from jax.experimental import pallas as pl
from jax.experimental.pallas import tpu as pltpu
from jax.experimental.pallas import tpu_sc as plsc

_NUM_VN = 6
_INV2T2 = 50.0
_BIG = 3.0e38

_NPAD = 30720
_WPTS = 960
_CH = 480
_NV = _CH // 16


def _transforms_kernel(x4_ref, w0, w1, w2, w3, w4, w5, out_ref):
    x = x4_ref[...]
    feats = [x]
    for f in (1.0, 2.0):
        feats.append(jnp.sin(x * f))
        feats.append(jnp.cos(x * f))
    h = jnp.concatenate(feats, axis=-1)
    for w in (w0, w1, w2, w3, w4):
        h = jnp.maximum(jnp.dot(h, w[...], preferred_element_type=jnp.float32), 0.0)
    o = jnp.dot(h, w5[...], preferred_element_type=jnp.float32)
    ax, ay, az = o[:, 0:1], o[:, 1:2], o[:, 2:3]
    x2, y2, z2 = ax * ax, ay * ay, az * az
    theta2 = x2 + y2 + z2
    theta = jnp.sqrt(theta2 + 1e-12)
    A = jnp.sin(theta) / theta
    B = (1.0 - jnp.cos(theta)) / (theta2 + 1e-12)
    xy, xz, yz = ax * ay, ax * az, ay * az
    r00 = 1.0 - B * (y2 + z2)
    r01 = -A * az + B * xy
    r02 = A * ay + B * xz
    r10 = A * az + B * xy
    r11 = 1.0 - B * (x2 + z2)
    r12 = -A * ax + B * yz
    r20 = -A * ay + B * xz
    r21 = A * ax + B * yz
    r22 = 1.0 - B * (x2 + y2)
    out_ref[...] = jnp.concatenate(
        [r00, r01, r02, r10, r11, r12, r20, r21, r22,
         o[:, 3:4], o[:, 4:5], o[:, 5:6]], axis=-1)


def _knn_kernel(ppt_ref, c2_ref, idx_ref, w_ref):
    e = jnp.dot(c2_ref[...], ppt_ref[...], preferred_element_type=jnp.float32)
    ncp, P = e.shape
    riota = lax.broadcasted_iota(jnp.int32, (ncp, P), 0).astype(jnp.float32)
    ew = e
    m1 = None
    idx_rows = []
    w_rows = []
    for k in range(_NUM_VN):
        mk = jnp.min(ew, axis=0, keepdims=True)
        if k == 0:
            m1 = mk
        t = jnp.where(ew <= mk, riota, 1.0e9)
        ik = jnp.min(t, axis=0, keepdims=True)
        idx_rows.append(ik)
        w_rows.append(jnp.exp((m1 - mk) * _INV2T2))
        if k < _NUM_VN - 1:
            ew = jnp.where(t == ik, _BIG, ew)
    z = w_rows[0]
    for k in range(1, _NUM_VN):
        z = z + w_rows[k]
    zero = jnp.zeros_like(m1)
    w8 = jnp.concatenate(w_rows + [zero, zero], axis=0) / z
    idx8 = jnp.concatenate(idx_rows + [zero, zero], axis=0)
    idx_ref[...] = idx8.astype(jnp.int32)
    w_ref[...] = w8


def _gsum(tab_v, ib, wk, comp):
    tv = tab_v.at[pl.ds(comp * 1000, 1000)]
    r = None
    for k in range(_NUM_VN):
        g = plsc.load_gather(tv, [ib[k]])
        r = g * wk[k] if r is None else r + g * wk[k]
    return r


def _sc_blend_body(tab_hbm, in_hbm, out_hbm, tab_v, in_v, out_v):
    wid = lax.axis_index("s") * 2 + lax.axis_index("c")
    pltpu.sync_copy(tab_hbm, tab_v)
    pltpu.sync_copy(in_hbm.at[pl.ds(wid * 15 * _WPTS, 15 * _WPTS)], in_v)
    for cc in range(_WPTS // _CH):
        cbase = cc * _CH

        @plsc.parallel_loop(0, _NV, unroll=1)
        def body(i):
            off = cbase + i * 16
            ib = [in_v[pl.ds(k * _WPTS + off, 16)] for k in range(_NUM_VN)]
            wk = [plsc.bitcast(in_v[pl.ds((6 + k) * _WPTS + off, 16)], jnp.float32)
                  for k in range(_NUM_VN)]
            pv = [plsc.bitcast(in_v[pl.ds((12 + d) * _WPTS + off, 16)], jnp.float32)
                  for d in range(3)]
            off_o = i * 16
            for b in range(8):
                for c3 in range(3):
                    r = _gsum(tab_v, ib, wk, b * 12 + 9 + c3)
                    for dd in range(3):
                        r = r + _gsum(tab_v, ib, wk, b * 12 + 3 * c3 + dd) * pv[dd]
                    out_v[pl.ds((b * 3 + c3) * _CH + off_o, 16)] = r

        pltpu.sync_copy(
            out_v, out_hbm.at[pl.ds((wid * 2 + cc) * 24 * _CH, 24 * _CH)])


def _make_sc_blend():
    return functools.partial(
        pl.kernel,
        mesh=plsc.VectorSubcoreMesh(core_axis_name="c", subcore_axis_name="s"),
        out_type=jax.ShapeDtypeStruct((32 * 2 * 24 * _CH,), jnp.float32),
        compiler_params=pltpu.CompilerParams(needs_layout_passes=False),
        scratch_types=[
            pltpu.VMEM((96000,), jnp.float32),
            pltpu.VMEM((15 * _WPTS,), jnp.int32),
            pltpu.VMEM((24 * _CH,), jnp.float32),
        ],
    )(_sc_blend_body)


def kernel(p, t, control_points, W0, W1, W2, W3, W4, W5):
    n = p.shape[0]
    b = t.shape[0]
    ncp = control_points.shape[0]

    tcol = jnp.repeat(t, ncp)[:, None]
    cps = jnp.tile(control_points, (b, 1))
    x4 = jnp.concatenate([tcol, cps], axis=1)
    R = 1000
    tab12 = pl.pallas_call(
        _transforms_kernel,
        grid=(b * ncp // R,),
        in_specs=[
            pl.BlockSpec((R, 4), lambda i: (i, 0)),
            pl.BlockSpec(W0.shape, lambda i: (0, 0)),
            pl.BlockSpec(W1.shape, lambda i: (0, 0)),
            pl.BlockSpec(W2.shape, lambda i: (0, 0)),
            pl.BlockSpec(W3.shape, lambda i: (0, 0)),
            pl.BlockSpec(W4.shape, lambda i: (0, 0)),
            pl.BlockSpec(W5.shape, lambda i: (0, 0)),
        ],
        out_specs=pl.BlockSpec((R, 12), lambda i: (i, 0)),
        out_shape=jax.ShapeDtypeStruct((b * ncp, 12), jnp.float32),
    )(x4, W0, W1, W2, W3, W4, W5)
    table = tab12.reshape(b, ncp, 12).transpose(1, 0, 2).reshape(ncp, b * 12)

    pad = _NPAD - n
    pt_pad = jnp.pad(p.T, ((0, 0), (0, pad)))
    ppt = jnp.concatenate(
        [pt_pad, jnp.ones((1, _NPAD), jnp.float32),
         jnp.zeros((4, _NPAD), jnp.float32)],
        axis=0)
    csq = jnp.sum(control_points * control_points, axis=1, keepdims=True)
    C2 = jnp.concatenate(
        [-2.0 * control_points, csq, jnp.zeros((ncp, 4), jnp.float32)],
        axis=1)

    P = 1536
    idx8, w8 = pl.pallas_call(
        _knn_kernel,
        grid=(_NPAD // P,),
        in_specs=[
            pl.BlockSpec((8, P), lambda i: (0, i)),
            pl.BlockSpec((ncp, 8), lambda i: (0, 0)),
        ],
        out_specs=[
            pl.BlockSpec((8, P), lambda i: (0, i)),
            pl.BlockSpec((8, P), lambda i: (0, i)),
        ],
        out_shape=[
            jax.ShapeDtypeStruct((8, _NPAD), jnp.int32),
            jax.ShapeDtypeStruct((8, _NPAD), jnp.float32),
        ],
    )(ppt, C2)

    comb = jnp.concatenate(
        [idx8[:6],
         lax.bitcast_convert_type(w8[:6], jnp.int32),
         lax.bitcast_convert_type(pt_pad, jnp.int32)], axis=0)
    in_flat = comb.reshape(15, 32, _WPTS).transpose(1, 0, 2).reshape(-1)
    tab_flat = table.T.reshape(-1)
    out_sc = _make_sc_blend()(tab_flat, in_flat)
    out = out_sc.reshape(32 * 2, 24, _CH).transpose(1, 0, 2).reshape(24, _NPAD)
    return out[:, :n].reshape(b, 3, n).transpose(0, 2, 1)
